# Initial kernel scaffold; baseline (speedup 1.0000x reference)
import functools

import jax
import jax.numpy as jnp
from jax import lax
from jax.experimental import pallas as pl
from jax.experimental.pallas import tpu as pltpu
from jax.experimental.pallas import tpu_sc as plsc

_NC = 2
_NS = 16
_L = 16
_F = 128
_K = 128
_DUM = 64


def _make_pool(n_edges, n_seg, n_src):
    shard = n_seg // _NC
    edges_per_tile = n_edges // _NS
    n_chunks = edges_per_tile // _K
    acc_rows = shard + _DUM
    zr = shard // _NS

    mesh = plsc.VectorSubcoreMesh(core_axis_name="c", subcore_axis_name="s")

    @functools.partial(
        pl.kernel,
        out_type=jax.ShapeDtypeStruct((n_seg, _F), jnp.float32),
        mesh=mesh,
        scratch_types=[
            pltpu.VMEM((_K, _F), jnp.float32),
            pltpu.VMEM((_K,), jnp.int32),
            pltpu.VMEM((_K,), jnp.int32),
            pltpu.VMEM((_K,), jnp.int32),
            pltpu.VMEM((_K,), jnp.float32),
            pltpu.VMEM_SHARED((acc_rows, _F), jnp.float32),
            pltpu.SemaphoreType.DMA,
        ],
    )
    def pool(src_hbm, vals_hbm, gidx_hbm, seg_hbm, out_hbm,
             rows, gidx, sidx, segv, valsv, acc, sem):
        c = lax.axis_index("c")
        s = lax.axis_index("s")
        lo = c * shard
        hi = lo + shard
        iota = lax.iota(jnp.int32, _L)
        zvec = jnp.zeros((_L,), jnp.float32)

        def zero_rows(e, t):
            for f in range(_F // _L):
                rows[e, pl.ds(f * _L, _L)] = zvec
            return t
        lax.fori_loop(0, _K, zero_rows, 0)
        for off in range(0, zr, _K):
            sz = min(_K, zr - off)
            pltpu.sync_copy(rows.at[pl.ds(0, sz)],
                            acc.at[pl.ds(s * zr + off, sz)])
        plsc.subcore_barrier()

        def chunk(i, t):
            base = s * edges_per_tile + i * _K
            pltpu.sync_copy(seg_hbm.at[pl.ds(base, _K)], segv)
            first = jnp.min(segv[pl.ds(0, _L)])
            last = jnp.max(segv[pl.ds(_K - _L, _L)])

            @pl.when(jnp.logical_and(last >= lo, first < hi))
            def _():
                pltpu.sync_copy(gidx_hbm.at[pl.ds(base, _K)], gidx)
                pltpu.sync_copy(vals_hbm.at[pl.ds(base, _K)], valsv)
                pltpu.async_copy(src_hbm.at[gidx], rows, sem).wait()

                def mk_sidx(g, t2):
                    sg = segv[pl.ds(g * _L, _L)]
                    loc = sg - lo
                    inb = jnp.logical_and(sg >= lo, sg < hi)
                    dummy = shard + ((g * _L + s * _L + iota) & (_DUM - 1))
                    sidx[pl.ds(g * _L, _L)] = jnp.where(inb, loc, dummy)
                    return t2
                lax.fori_loop(0, _K // _L, mk_sidx, 0)

                def scale_group(g, t2):
                    vv = valsv[pl.ds(g * _L, _L)]
                    for j in range(_L):
                        e = g * _L + j
                        vs = jnp.sum(jnp.where(iota == j, vv, 0.0))
                        for f in range(_F // _L):
                            rows[e, pl.ds(f * _L, _L)] = (
                                rows[e, pl.ds(f * _L, _L)] * vs)
                    return t2
                lax.fori_loop(0, _K // _L, scale_group, 0)

                pltpu.sync_copy(rows, acc.at[sidx], add=True)
            return t
        lax.fori_loop(0, n_chunks, chunk, 0)

        plsc.subcore_barrier()
        for off in range(0, zr, _K):
            sz = min(_K, zr - off)
            pltpu.sync_copy(acc.at[pl.ds(s * zr + off, sz)],
                            out_hbm.at[pl.ds(lo + s * zr + off, sz)])

    return pool


_pool_t2m = _make_pool(32768, 8192, 32768)
_pool_m2c = _make_pool(8192, 4096, 8192)


def kernel(token_features, t2m_vals, m2c_vals, t2m_token_idx,
           t2m_mention_idx, m2c_mention_idx, m2c_concept_idx):
    tok_idx = t2m_token_idx.astype(jnp.int32)
    men_idx = t2m_mention_idx.astype(jnp.int32)
    m2c_men = m2c_mention_idx.astype(jnp.int32)
    con_idx = m2c_concept_idx.astype(jnp.int32)
    mentions = _pool_t2m(token_features, t2m_vals, tok_idx, men_idx)
    concepts = _pool_m2c(mentions, m2c_vals, m2c_men, con_idx)
    return (mentions, concepts)


# speedup vs baseline: 2.0615x; 2.0615x over previous
import functools

import jax
import jax.numpy as jnp
from jax import lax
from jax.experimental import pallas as pl
from jax.experimental.pallas import tpu as pltpu
from jax.experimental.pallas import tpu_sc as plsc

_NC = 2
_NS = 16
_L = 16
_F = 128
_K = 128
_DUM = 64


def _make_pool(n_edges, n_seg, n_src):
    shard = n_seg // _NC
    edges_per_tile = n_edges // _NS
    n_chunks = edges_per_tile // _K
    acc_rows = shard + _DUM
    zr = shard // _NS

    mesh = plsc.VectorSubcoreMesh(core_axis_name="c", subcore_axis_name="s")

    @functools.partial(
        pl.kernel,
        out_type=jax.ShapeDtypeStruct((n_seg, _F), jnp.float32),
        mesh=mesh,
        compiler_params=pltpu.CompilerParams(needs_layout_passes=False),
        scratch_types=[
            pltpu.VMEM((_K, _F), jnp.float32),
            pltpu.VMEM((_K,), jnp.int32),
            pltpu.VMEM((_K,), jnp.int32),
            pltpu.VMEM((_K,), jnp.int32),
            pltpu.VMEM((_K,), jnp.float32),
            pltpu.VMEM_SHARED((acc_rows, _F), jnp.float32),
            pltpu.SemaphoreType.DMA,
        ],
    )
    def pool(src_hbm, vals_hbm, gidx_hbm, seg_hbm, out_hbm,
             rows, gidx, sidx, segv, valsv, acc, sem):
        c = lax.axis_index("c")
        s = lax.axis_index("s")
        lo = c * shard
        hi = lo + shard
        iota = lax.iota(jnp.int32, _L)
        zvec = jnp.zeros((_L,), jnp.float32)

        def zero_rows(e, t):
            for f in range(_F // _L):
                rows[e, pl.ds(f * _L, _L)] = zvec
            return t
        lax.fori_loop(0, _K, zero_rows, 0)
        for off in range(0, zr, _K):
            sz = min(_K, zr - off)
            pltpu.sync_copy(rows.at[pl.ds(0, sz)],
                            acc.at[pl.ds(s * zr + off, sz)])
        plsc.subcore_barrier()

        def chunk(i, t):
            base = s * edges_per_tile + i * _K
            pltpu.sync_copy(seg_hbm.at[pl.ds(base, _K)], segv)
            pltpu.sync_copy(gidx_hbm.at[pl.ds(base, _K)], gidx)
            pltpu.sync_copy(vals_hbm.at[pl.ds(base, _K)], valsv)
            pltpu.async_copy(src_hbm.at[gidx], rows, sem).wait()

            def mk_sidx(g, t2):
                sg = segv[pl.ds(g * _L, _L)]
                loc = sg - lo
                inb = jnp.logical_and(sg >= lo, sg < hi)
                dummy = shard + ((g * _L + s * _L + iota) & (_DUM - 1))
                sidx[pl.ds(g * _L, _L)] = jnp.where(inb, loc, dummy)
                return t2
            lax.fori_loop(0, _K // _L, mk_sidx, 0)

            def scale_edge(e, t2):
                vs = plsc.load_gather(valsv, [jnp.full((_L,), 0, jnp.int32) + e])
                for f in range(_F // _L):
                    rows[e, pl.ds(f * _L, _L)] = (
                        rows[e, pl.ds(f * _L, _L)] * vs)
                return t2
            lax.fori_loop(0, _K, scale_edge, 0)

            pltpu.sync_copy(rows, acc.at[sidx], add=True)
            return t
        lax.fori_loop(0, n_chunks, chunk, 0)

        plsc.subcore_barrier()
        for off in range(0, zr, _K):
            sz = min(_K, zr - off)
            pltpu.sync_copy(acc.at[pl.ds(s * zr + off, sz)],
                            out_hbm.at[pl.ds(lo + s * zr + off, sz)])

    return pool


_pool_t2m = _make_pool(32768, 8192, 32768)
_pool_m2c = _make_pool(8192, 4096, 8192)


def kernel(token_features, t2m_vals, m2c_vals, t2m_token_idx,
           t2m_mention_idx, m2c_mention_idx, m2c_concept_idx):
    tok_idx = t2m_token_idx.astype(jnp.int32)
    men_idx = t2m_mention_idx.astype(jnp.int32)
    m2c_men = m2c_mention_idx.astype(jnp.int32)
    con_idx = m2c_concept_idx.astype(jnp.int32)
    mentions = _pool_t2m(token_features, t2m_vals, tok_idx, men_idx)
    concepts = _pool_m2c(mentions, m2c_vals, m2c_men, con_idx)
    return (mentions, concepts)


# speedup vs baseline: 2.9433x; 1.4277x over previous
import functools

import jax
import jax.numpy as jnp
from jax import lax
from jax.experimental import pallas as pl
from jax.experimental.pallas import tpu as pltpu
from jax.experimental.pallas import tpu_sc as plsc

_NC = 2
_NS = 16
_L = 16
_F = 128
_K = 128
_DUM = 64


def _make_pool(n_edges, n_seg, n_src):
    shard = n_seg // _NC
    edges_per_tile = n_edges // _NS
    n_chunks = edges_per_tile // _K
    acc_rows = shard + _DUM
    zr = shard // _NS

    mesh = plsc.VectorSubcoreMesh(core_axis_name="c", subcore_axis_name="s")

    @functools.partial(
        pl.kernel,
        out_type=jax.ShapeDtypeStruct((n_seg, _F), jnp.float32),
        mesh=mesh,
        compiler_params=pltpu.CompilerParams(needs_layout_passes=False),
        scratch_types=[
            pltpu.VMEM((_K, _F), jnp.float32),
            pltpu.VMEM((_K,), jnp.int32),
            pltpu.VMEM((_K,), jnp.int32),
            pltpu.VMEM((_K,), jnp.int32),
            pltpu.VMEM((_K,), jnp.float32),
            pltpu.VMEM_SHARED((acc_rows, _F), jnp.float32),
            pltpu.SemaphoreType.DMA,
        ],
    )
    def pool(src_hbm, vals_hbm, gidx_hbm, seg_hbm, out_hbm,
             rows, gidx, sidx, segv, valsv, acc, sem):
        c = lax.axis_index("c")
        s = lax.axis_index("s")
        lo = c * shard
        hi = lo + shard
        iota = lax.iota(jnp.int32, _L)
        zvec = jnp.zeros((_L,), jnp.float32)

        def zero_rows(e, t):
            for f in range(_F // _L):
                rows[e, pl.ds(f * _L, _L)] = zvec
            return t
        lax.fori_loop(0, _K, zero_rows, 0)
        for off in range(0, zr, _K):
            sz = min(_K, zr - off)
            pltpu.sync_copy(rows.at[pl.ds(0, sz)],
                            acc.at[pl.ds(s * zr + off, sz)])
        plsc.subcore_barrier()

        def chunk(i, t):
            base = (i * _NS + s) * _K
            pltpu.sync_copy(seg_hbm.at[pl.ds(base, _K)], segv)
            first = jnp.min(segv[pl.ds(0, _L)])
            last = jnp.max(segv[pl.ds(_K - _L, _L)])

            @pl.when(jnp.logical_and(last >= lo, first < hi))
            def _():
                pltpu.sync_copy(gidx_hbm.at[pl.ds(base, _K)], gidx)
                pltpu.sync_copy(vals_hbm.at[pl.ds(base, _K)], valsv)
                pltpu.async_copy(src_hbm.at[gidx], rows, sem).wait()

                def mk_sidx(g, t2):
                    sg = segv[pl.ds(g * _L, _L)]
                    loc = sg - lo
                    inb = jnp.logical_and(sg >= lo, sg < hi)
                    dummy = shard + ((g * _L + s * _L + iota) & (_DUM - 1))
                    sidx[pl.ds(g * _L, _L)] = jnp.where(inb, loc, dummy)
                    return t2
                lax.fori_loop(0, _K // _L, mk_sidx, 0)

                def scale_edge(e, t2):
                    vs = plsc.load_gather(
                        valsv, [jnp.full((_L,), 0, jnp.int32) + e])
                    for f in range(_F // _L):
                        rows[e, pl.ds(f * _L, _L)] = (
                            rows[e, pl.ds(f * _L, _L)] * vs)
                    return t2
                lax.fori_loop(0, _K, scale_edge, 0)

                pltpu.sync_copy(rows, acc.at[sidx], add=True)
            return t
        lax.fori_loop(0, n_chunks, chunk, 0)

        plsc.subcore_barrier()
        for off in range(0, zr, _K):
            sz = min(_K, zr - off)
            pltpu.sync_copy(acc.at[pl.ds(s * zr + off, sz)],
                            out_hbm.at[pl.ds(lo + s * zr + off, sz)])

    return pool


_pool_t2m = _make_pool(32768, 8192, 32768)
_pool_m2c = _make_pool(8192, 4096, 8192)


def kernel(token_features, t2m_vals, m2c_vals, t2m_token_idx,
           t2m_mention_idx, m2c_mention_idx, m2c_concept_idx):
    tok_idx = t2m_token_idx.astype(jnp.int32)
    men_idx = t2m_mention_idx.astype(jnp.int32)
    m2c_men = m2c_mention_idx.astype(jnp.int32)
    con_idx = m2c_concept_idx.astype(jnp.int32)
    mentions = _pool_t2m(token_features, t2m_vals, tok_idx, men_idx)
    concepts = _pool_m2c(mentions, m2c_vals, m2c_men, con_idx)
    return (mentions, concepts)


# speedup vs baseline: 3.1466x; 1.0691x over previous
import functools

import jax
import jax.numpy as jnp
from jax import lax
from jax.experimental import pallas as pl
from jax.experimental.pallas import tpu as pltpu
from jax.experimental.pallas import tpu_sc as plsc

_NC = 2
_NS = 16
_L = 16
_F = 128
_K = 128
_DUM = 64


def _make_pool(n_edges, n_seg, n_src):
    shard = n_seg // _NC
    edges_per_tile = n_edges // _NS
    n_chunks = edges_per_tile // _K
    acc_rows = shard + _DUM
    zr = shard // _NS

    mesh = plsc.VectorSubcoreMesh(core_axis_name="c", subcore_axis_name="s")

    @functools.partial(
        pl.kernel,
        out_type=jax.ShapeDtypeStruct((n_seg, _F), jnp.float32),
        mesh=mesh,
        compiler_params=pltpu.CompilerParams(needs_layout_passes=False),
        scratch_types=[
            pltpu.VMEM((_K, _F), jnp.float32),
            pltpu.VMEM((_K, _F), jnp.float32),
            pltpu.VMEM((_K,), jnp.int32),
            pltpu.VMEM((_K,), jnp.int32),
            pltpu.VMEM((_K,), jnp.int32),
            pltpu.VMEM((_K,), jnp.int32),
            pltpu.VMEM((_K,), jnp.int32),
            pltpu.VMEM((_K,), jnp.int32),
            pltpu.VMEM((_K,), jnp.float32),
            pltpu.VMEM((_K,), jnp.float32),
            pltpu.VMEM_SHARED((acc_rows, _F), jnp.float32),
            pltpu.SemaphoreType.DMA,
            pltpu.SemaphoreType.DMA,
        ],
    )
    def pool(src_hbm, vals_hbm, gidx_hbm, seg_hbm, out_hbm,
             rows0, rows1, gidx0, gidx1, sidx0, sidx1, segv0, segv1,
             valsv0, valsv1, acc, sem0, sem1):
        c = lax.axis_index("c")
        s = lax.axis_index("s")
        lo = c * shard
        hi = lo + shard
        iota = lax.iota(jnp.int32, _L)
        zvec = jnp.zeros((_L,), jnp.float32)

        def zero_rows(e, t):
            for f in range(_F // _L):
                rows0[e, pl.ds(f * _L, _L)] = zvec
            return t
        lax.fori_loop(0, _K, zero_rows, 0)
        for off in range(0, zr, _K):
            sz = min(_K, zr - off)
            pltpu.sync_copy(rows0.at[pl.ds(0, sz)],
                            acc.at[pl.ds(s * zr + off, sz)])
        plsc.subcore_barrier()

        def load_small(i, segb, gidxb, valsb):
            base = (i * _NS + s) * _K
            pltpu.sync_copy(seg_hbm.at[pl.ds(base, _K)], segb)
            pltpu.sync_copy(gidx_hbm.at[pl.ds(base, _K)], gidxb)
            pltpu.sync_copy(vals_hbm.at[pl.ds(base, _K)], valsb)

        def cond_of(segb):
            return jnp.logical_and(
                jnp.max(segb[pl.ds(_K - _L, _L)]) >= lo,
                jnp.min(segb[pl.ds(0, _L)]) < hi)

        def process(segb, gidxb, valsb, rowsb, sidxb, semb):
            @pl.when(cond_of(segb))
            def _():
                pltpu.make_async_copy(src_hbm.at[gidxb], rowsb, semb).wait()

                def mk_sidx(g, t2):
                    sg = segb[pl.ds(g * _L, _L)]
                    loc = sg - lo
                    inb = jnp.logical_and(sg >= lo, sg < hi)
                    dummy = shard + ((g * _L + s * _L + iota) & (_DUM - 1))
                    sidxb[pl.ds(g * _L, _L)] = jnp.where(inb, loc, dummy)
                    return t2
                lax.fori_loop(0, _K // _L, mk_sidx, 0)

                def scale_edge(e2, t2):
                    for u in range(2):
                        e = e2 * 2 + u
                        vs = plsc.load_gather(
                            valsb, [jnp.full((_L,), 0, jnp.int32) + e])
                        for f in range(_F // _L):
                            rowsb[e, pl.ds(f * _L, _L)] = (
                                rowsb[e, pl.ds(f * _L, _L)] * vs)
                    return t2
                lax.fori_loop(0, _K // 2, scale_edge, 0)

                pltpu.sync_copy(rowsb, acc.at[sidxb], add=True)

        load_small(0, segv0, gidx0, valsv0)

        @pl.when(cond_of(segv0))
        def _prime():
            pltpu.async_copy(src_hbm.at[gidx0], rows0, sem0)

        def pair(k, t):
            i0 = 2 * k
            load_small(i0 + 1, segv1, gidx1, valsv1)

            @pl.when(cond_of(segv1))
            def _g1():
                pltpu.async_copy(src_hbm.at[gidx1], rows1, sem1)

            process(segv0, gidx0, valsv0, rows0, sidx0, sem0)

            @pl.when(k < n_chunks // 2 - 1)
            def _next0():
                load_small(i0 + 2, segv0, gidx0, valsv0)

                @pl.when(cond_of(segv0))
                def _g0():
                    pltpu.async_copy(src_hbm.at[gidx0], rows0, sem0)

            process(segv1, gidx1, valsv1, rows1, sidx1, sem1)
            return t
        lax.fori_loop(0, n_chunks // 2, pair, 0)

        plsc.subcore_barrier()
        for off in range(0, zr, _K):
            sz = min(_K, zr - off)
            pltpu.sync_copy(acc.at[pl.ds(s * zr + off, sz)],
                            out_hbm.at[pl.ds(lo + s * zr + off, sz)])

    return pool


_pool_t2m = _make_pool(32768, 8192, 32768)
_pool_m2c = _make_pool(8192, 4096, 8192)


def kernel(token_features, t2m_vals, m2c_vals, t2m_token_idx,
           t2m_mention_idx, m2c_mention_idx, m2c_concept_idx):
    tok_idx = t2m_token_idx.astype(jnp.int32)
    men_idx = t2m_mention_idx.astype(jnp.int32)
    m2c_men = m2c_mention_idx.astype(jnp.int32)
    con_idx = m2c_concept_idx.astype(jnp.int32)
    mentions = _pool_t2m(token_features, t2m_vals, tok_idx, men_idx)
    concepts = _pool_m2c(mentions, m2c_vals, m2c_men, con_idx)
    return (mentions, concepts)


# speedup vs baseline: 4.3828x; 1.3929x over previous
import functools

import jax
import jax.numpy as jnp
from jax import lax
from jax.experimental import pallas as pl
from jax.experimental.pallas import tpu as pltpu
from jax.experimental.pallas import tpu_sc as plsc

_NC = 2
_NS = 16
_L = 16
_F = 128
_K = 128
_DUM = 64
_NBUF = 3


def _make_pool(n_edges, n_seg, n_src):
    shard = n_seg // _NC
    ept = n_edges // _NS
    blk = ept // 2
    half = n_edges // 2
    n_chunks = ept // _K
    cpb = blk // _K
    acc_rows = shard + _DUM
    zr = shard // _NS

    mesh = plsc.VectorSubcoreMesh(core_axis_name="c", subcore_axis_name="s")

    @functools.partial(
        pl.kernel,
        out_type=jax.ShapeDtypeStruct((n_seg, _F), jnp.float32),
        mesh=mesh,
        compiler_params=pltpu.CompilerParams(needs_layout_passes=False),
        scratch_types=[
            [pltpu.VMEM((_K, _F), jnp.float32) for _ in range(_NBUF)],
            [pltpu.VMEM((_K,), jnp.int32) for _ in range(_NBUF)],
            pltpu.VMEM((ept,), jnp.int32),
            pltpu.VMEM((ept,), jnp.int32),
            pltpu.VMEM((ept,), jnp.float32),
            pltpu.VMEM_SHARED((acc_rows, _F), jnp.float32),
            [pltpu.SemaphoreType.DMA for _ in range(_NBUF)],
            [pltpu.SemaphoreType.DMA for _ in range(_NBUF)],
            pltpu.SemaphoreType.DMA,
        ],
    )
    def pool(src_hbm, vals_hbm, gidx_hbm, seg_hbm, out_hbm,
             rows, sidx, segall, gidxall, valsall, acc, gsem, ssem, insem):
        c = lax.axis_index("c")
        s = lax.axis_index("s")
        lo = c * shard
        hi = lo + shard
        iota = lax.iota(jnp.int32, _L)
        zvec = jnp.zeros((_L,), jnp.float32)

        stage = []
        for b in range(2):
            hbase = b * half + s * blk
            vbase = b * blk
            for src, dst in ((seg_hbm, segall), (gidx_hbm, gidxall),
                             (vals_hbm, valsall)):
                stage.append((src.at[pl.ds(hbase, blk)], dst.at[pl.ds(vbase, blk)]))
        for src, dst in stage:
            pltpu.async_copy(src, dst, insem)

        def zero_rows(e, t):
            for f in range(_F // _L):
                rows[0][e, pl.ds(f * _L, _L)] = zvec
            return t
        lax.fori_loop(0, _K, zero_rows, 0)
        for off in range(0, zr, _K):
            sz = min(_K, zr - off)
            pltpu.sync_copy(rows[0].at[pl.ds(0, sz)],
                            acc.at[pl.ds(s * zr + off, sz)])
        plsc.subcore_barrier()
        for src, dst in stage:
            pltpu.make_async_copy(src, dst, insem).wait()

        def cond_of(ci):
            off = ci * _K
            return jnp.logical_and(
                jnp.max(segall[pl.ds(off + _K - _L, _L)]) >= lo,
                jnp.min(segall[pl.ds(off, _L)]) < hi)

        def gather_desc(ci, b):
            return pltpu.make_async_copy(
                src_hbm.at[gidxall.at[pl.ds(ci * _K, _K)]], rows[b], gsem[b])

        def scatter_desc(b):
            return pltpu.make_async_copy(rows[b], acc.at[sidx[b]], ssem[b])

        def process(ci, b):
            @pl.when(cond_of(ci))
            def _():
                gather_desc(ci, b).wait()
                off = ci * _K

                def mk_sidx(g, t2):
                    sg = segall[pl.ds(off + g * _L, _L)]
                    loc = sg - lo
                    inb = jnp.logical_and(sg >= lo, sg < hi)
                    dummy = shard + ((g * _L + s * _L + iota) & (_DUM - 1))
                    sidx[b][pl.ds(g * _L, _L)] = jnp.where(inb, loc, dummy)
                    return t2
                lax.fori_loop(0, _K // _L, mk_sidx, 0)

                def scale_edge(e2, t2):
                    for u in range(2):
                        e = e2 * 2 + u
                        vs = plsc.load_gather(
                            valsall, [jnp.full((_L,), off, jnp.int32) + e])
                        for f in range(_F // _L):
                            rows[b][e, pl.ds(f * _L, _L)] = (
                                rows[b][e, pl.ds(f * _L, _L)] * vs)
                    return t2
                lax.fori_loop(0, _K // 2, scale_edge, 0)

                scatter_desc(b).start(add=True)

        for ci in range(min(2, n_chunks)):
            @pl.when(cond_of(ci))
            def _(ci=ci):
                gather_desc(ci, ci % _NBUF).start()
        for ci in range(n_chunks):
            b = ci % _NBUF
            process(ci, b)
            nxt = ci + 2
            if nxt < n_chunks:
                bn = nxt % _NBUF
                prev = nxt - _NBUF
                if prev >= 0:
                    @pl.when(cond_of(prev))
                    def _(prev=prev, bn=bn):
                        scatter_desc(bn).wait()

                @pl.when(cond_of(nxt))
                def _(nxt=nxt, bn=bn):
                    gather_desc(nxt, bn).start()
        for ci in range(max(0, n_chunks - _NBUF), n_chunks):
            @pl.when(cond_of(ci))
            def _(ci=ci):
                scatter_desc(ci % _NBUF).wait()

        plsc.subcore_barrier()
        for off in range(0, zr, _K):
            sz = min(_K, zr - off)
            pltpu.sync_copy(acc.at[pl.ds(s * zr + off, sz)],
                            out_hbm.at[pl.ds(lo + s * zr + off, sz)])

    return pool


_pool_t2m = _make_pool(32768, 8192, 32768)
_pool_m2c = _make_pool(8192, 4096, 8192)


def kernel(token_features, t2m_vals, m2c_vals, t2m_token_idx,
           t2m_mention_idx, m2c_mention_idx, m2c_concept_idx):
    tok_idx = t2m_token_idx.astype(jnp.int32)
    men_idx = t2m_mention_idx.astype(jnp.int32)
    m2c_men = m2c_mention_idx.astype(jnp.int32)
    con_idx = m2c_concept_idx.astype(jnp.int32)
    mentions = _pool_t2m(token_features, t2m_vals, tok_idx, men_idx)
    concepts = _pool_m2c(mentions, m2c_vals, m2c_men, con_idx)
    return (mentions, concepts)
